# baseline (speedup 1.0000x reference)
import functools

import jax
import jax.numpy as jnp
from jax.experimental import pallas as pl
from jax.experimental.pallas import tpu as pltpu

G_SIGMA = 1.0
_C = 256.0


def _lrelu(v):
    return jnp.where(v >= 0, v, 0.2 * v)


def _elu(v):
    return jnp.where(v > 0, v, jnp.exp(v) - 1.0)


def _relation(xr, mean, wx_ref, wm_ref, m_ref, w_ref, fac):
    f = xr.shape[1]
    gb = (jnp.dot(xr, wx_ref[...], preferred_element_type=jnp.float32)
          + jnp.dot(mean, wm_ref[...], preferred_element_type=jnp.float32))
    gamma = _lrelu(gb[:, :f]) + 1.0
    beta = _lrelu(gb[:, f:])
    miss = xr + gamma * m_ref[...] + beta - mean
    h = mean + fac * miss
    out = jnp.dot(h, w_ref[...], preferred_element_type=jnp.float32)
    return out, miss


def _layer1_body(br, brp, adj_ref, xf_ref, wx_ref, wm_ref, m_ref, w_ref,
                 fac_ref, out_ref, miss_ref, q_ref, s_ref):
    i = pl.program_id(0)
    n = adj_ref.shape[1]
    adjb = adj_ref[...]
    mean = jnp.dot(adjb, xf_ref[...], preferred_element_type=jnp.float32)
    rmax = jnp.maximum(jnp.max(jnp.abs(adjb), axis=1, keepdims=True), 1e-30)
    scaled = jnp.concatenate(
        [adjb * (_C / rmax), jnp.zeros((brp - br, n), jnp.float32)], axis=0)
    q_ref[...] = scaled.astype(jnp.float8_e4m3fn)
    s_ref[...] = rmax * (1.0 / _C)
    xr = xf_ref[pl.ds(i * br, br), :]
    out, miss = _relation(xr, mean, wx_ref, wm_ref, m_ref, w_ref, fac_ref[0])
    out_ref[...] = _elu(out)
    miss_ref[...] = miss


def _layer2_body(br, q_ref, s_ref, xf_ref, wx_ref, wm_ref, m_ref, w_ref,
                 fac_ref, out_ref, miss_ref, lsm_ref, hq_ref, lq_ref, sx_ref):
    i = pl.program_id(0)

    @pl.when(i == 0)
    def _quantize_x1():
        v = xf_ref[...]
        sx = jnp.maximum(jnp.max(jnp.abs(v)), 1e-30) * (1.0 / _C)
        vi = v * (1.0 / sx)
        hq = vi.astype(jnp.float8_e4m3fn)
        hq_ref[...] = hq
        lq_ref[...] = ((vi - hq.astype(jnp.float32)) * 16.0).astype(
            jnp.float8_e4m3fn)
        sx_ref[...] = jnp.full((1, 1), sx, jnp.float32)

    qa = q_ref[...]
    acc_h = jnp.dot(qa, hq_ref[...], preferred_element_type=jnp.float32)
    acc_l = jnp.dot(qa, lq_ref[...], preferred_element_type=jnp.float32)
    sc = s_ref[...] * sx_ref[0, 0]
    mean = (acc_h[:br] + acc_l[:br] * (1.0 / 16.0)) * sc
    xr = xf_ref[pl.ds(i * br, br), :]
    out, miss = _relation(xr, mean, wx_ref, wm_ref, m_ref, w_ref, fac_ref[0])
    out_ref[...] = out
    miss_ref[...] = miss
    mx = jnp.max(out, axis=1, keepdims=True)
    sh = out - mx
    lse = jnp.log(jnp.sum(jnp.exp(sh), axis=1, keepdims=True))
    lsm_ref[...] = sh - lse


def _wspecs(f, fo):
    return [
        pl.BlockSpec((f, 2 * f), lambda i: (0, 0)),
        pl.BlockSpec((f, 2 * f), lambda i: (0, 0)),
        pl.BlockSpec((1, f), lambda i: (0, 0)),
        pl.BlockSpec((f, fo), lambda i: (0, 0)),
        pl.BlockSpec(memory_space=pltpu.SMEM),
    ]


def _params():
    return pltpu.CompilerParams(
        dimension_semantics=("parallel",),
        vmem_limit_bytes=110 * 1024 * 1024,
    )


def _layer1(x, adj, wx, wm, m, w, fac, br, brp):
    n, f = x.shape
    fo = w.shape[1]
    g = n // br
    return pl.pallas_call(
        functools.partial(_layer1_body, br, brp),
        grid=(g,),
        in_specs=[
            pl.BlockSpec((br, n), lambda i: (i, 0)),
            pl.BlockSpec((n, f), lambda i: (0, 0)),
        ] + _wspecs(f, fo),
        out_specs=[
            pl.BlockSpec((br, fo), lambda i: (i, 0)),
            pl.BlockSpec((br, f), lambda i: (i, 0)),
            pl.BlockSpec((brp, n), lambda i: (i, 0)),
            pl.BlockSpec((br, 1), lambda i: (i, 0)),
        ],
        out_shape=[
            jax.ShapeDtypeStruct((n, fo), jnp.float32),
            jax.ShapeDtypeStruct((n, f), jnp.float32),
            jax.ShapeDtypeStruct((g * brp, n), jnp.float8_e4m3fn),
            jax.ShapeDtypeStruct((n, 1), jnp.float32),
        ],
        compiler_params=_params(),
    )(adj, x, wx, wm, m, w, fac)


def _layer2(x1, q, s, wx, wm, m, w, fac, br, brp):
    n, f = x1.shape
    fo = w.shape[1]
    return pl.pallas_call(
        functools.partial(_layer2_body, br),
        grid=(n // br,),
        in_specs=[
            pl.BlockSpec((brp, n), lambda i: (i, 0)),
            pl.BlockSpec((br, 1), lambda i: (i, 0)),
            pl.BlockSpec((n, f), lambda i: (0, 0)),
        ] + _wspecs(f, fo),
        out_specs=[
            pl.BlockSpec((br, fo), lambda i: (i, 0)),
            pl.BlockSpec((br, f), lambda i: (i, 0)),
            pl.BlockSpec((br, fo), lambda i: (i, 0)),
        ],
        out_shape=[
            jax.ShapeDtypeStruct((n, fo), jnp.float32),
            jax.ShapeDtypeStruct((n, f), jnp.float32),
            jax.ShapeDtypeStruct((n, fo), jnp.float32),
        ],
        scratch_shapes=[
            pltpu.VMEM((n, f), jnp.float8_e4m3fn),
            pltpu.VMEM((n, f), jnp.float8_e4m3fn),
            pltpu.VMEM((1, 1), jnp.float32),
        ],
        compiler_params=pltpu.CompilerParams(
            dimension_semantics=("arbitrary",),
            vmem_limit_bytes=110 * 1024 * 1024,
        ),
    )(q, s, x1, wx, wm, m, w, fac)


def kernel(x, adj, head, r1_g1, r1_g2, r1_b1, r1_b2, r2_g1, r2_g2, r2_b1,
           r2_b2, r1_m, r2_m, r1_w, r2_w):
    n = x.shape[0]
    br = next(b for b in (400, 200, 80, 16, 8, 1) if n % b == 0)
    brp = ((br + 31) // 32) * 32
    fac = jnp.where(head != 0, 0.0, G_SIGMA).astype(jnp.float32).reshape(1)
    wx1 = jnp.concatenate([r1_g1, r1_b1], axis=1)
    wm1 = jnp.concatenate([r1_g2, r1_b2], axis=1)
    wx2 = jnp.concatenate([r2_g1, r2_b1], axis=1)
    wm2 = jnp.concatenate([r2_g2, r2_b2], axis=1)
    x1, out1, q, s = _layer1(x, adj, wx1, wm1, r1_m, r1_w, fac, br, brp)
    x2, out2, lsm = _layer2(x1, q, s, wx2, wm2, r2_m, r2_w, fac, br, brp)
    return x2, lsm, out1, out2


# speedup vs baseline: 1.0219x; 1.0219x over previous
import functools

import jax
import jax.numpy as jnp
from jax.experimental import pallas as pl
from jax.experimental.pallas import tpu as pltpu

G_SIGMA = 1.0
_C = 256.0


def _lrelu(v):
    return jnp.where(v >= 0, v, 0.2 * v)


def _elu(v):
    return jnp.where(v > 0, v, jnp.exp(v) - 1.0)


def _relation(xr, mean, wx_ref, wm_ref, m_ref, w_ref, fac):
    f = xr.shape[1]
    gb = (jnp.dot(xr, wx_ref[...], preferred_element_type=jnp.float32)
          + jnp.dot(mean, wm_ref[...], preferred_element_type=jnp.float32))
    gamma = _lrelu(gb[:, :f]) + 1.0
    beta = _lrelu(gb[:, f:])
    miss = xr + gamma * m_ref[...] + beta - mean
    h = mean + fac * miss
    out = jnp.dot(h, w_ref[...], preferred_element_type=jnp.float32)
    return out, miss


def _fused_body(br, brp, g,
                adj_ref, x_ref, wx1_ref, wm1_ref, m1_ref, w1_ref,
                wx2_ref, wm2_ref, m2_ref, w2_ref, fac_ref,
                x2_ref, lsm_ref, miss1_ref, miss2_ref, q_ref,
                qb0, qb1, x1_scr, s_scr, hq_ref, lq_ref,
                st_sem0, st_sem1, rd_sem0, rd_sem1):
    i = pl.program_id(0)
    n = adj_ref.shape[1]
    f = x_ref.shape[1]

    def q_stripe(k):
        return q_ref.at[pl.ds(k * brp, brp), :]

    @pl.when(i < g)
    def _phase_a():
        adjb = adj_ref[...]
        mean = jnp.dot(adjb, x_ref[...], preferred_element_type=jnp.float32)
        rmax = jnp.maximum(jnp.max(jnp.abs(adjb), axis=1, keepdims=True),
                           1e-30)
        scaled = jnp.concatenate(
            [adjb * (_C / rmax), jnp.zeros((brp - br, n), jnp.float32)],
            axis=0)
        qv = scaled.astype(jnp.float8_e4m3fn)
        s_scr[i] = jnp.transpose(rmax * (1.0 / _C))

        @pl.when(i % 2 == 0)
        def _even():
            @pl.when(i >= 2)
            def _():
                pltpu.make_async_copy(qb0, q_stripe(i - 2), st_sem0).wait()
            qb0[...] = qv
            pltpu.make_async_copy(qb0, q_stripe(i), st_sem0).start()

        @pl.when(i % 2 == 1)
        def _odd():
            @pl.when(i >= 2)
            def _():
                pltpu.make_async_copy(qb1, q_stripe(i - 2), st_sem1).wait()
            qb1[...] = qv
            pltpu.make_async_copy(qb1, q_stripe(i), st_sem1).start()

        xr = x_ref[pl.ds(i * br, br), :]
        out, miss = _relation(xr, mean, wx1_ref, wm1_ref, m1_ref, w1_ref,
                              fac_ref[0])
        miss1_ref[...] = miss
        x1_scr[pl.ds(i * br, br), :] = _elu(out)

    @pl.when(i >= g)
    def _phase_b():
        j = i - g

        @pl.when(j == 0)
        def _start_b():
            pltpu.make_async_copy(qb0, q_stripe(0), st_sem0).wait()
            if g >= 2:
                pltpu.make_async_copy(qb1, q_stripe(0), st_sem1).wait()
            v = x1_scr[...]
            sx = jnp.maximum(jnp.max(jnp.abs(v)), 1e-30) * (1.0 / _C)
            vi = v * (1.0 / sx)
            hq = vi.astype(jnp.float8_e4m3fn)
            hq_ref[...] = hq
            lq_ref[...] = ((vi - hq.astype(jnp.float32)) * 16.0).astype(
                jnp.float8_e4m3fn)
            s_scr[...] = s_scr[...] * sx
            pltpu.make_async_copy(q_stripe(0), qb0, rd_sem0).start()
            if g >= 2:
                pltpu.make_async_copy(q_stripe(1), qb1, rd_sem1).start()

        @pl.when((j > 0) & (j + 1 < g) & ((j + 1) % 2 == 0))
        def _pf_even():
            pltpu.make_async_copy(q_stripe(j + 1), qb0, rd_sem0).start()

        @pl.when((j > 0) & (j + 1 < g) & ((j + 1) % 2 == 1))
        def _pf_odd():
            pltpu.make_async_copy(q_stripe(j + 1), qb1, rd_sem1).start()

        def compute(qb_ref):
            qa = qb_ref[...]
            acc_h = jnp.dot(qa, hq_ref[...],
                            preferred_element_type=jnp.float32)
            acc_l = jnp.dot(qa, lq_ref[...],
                            preferred_element_type=jnp.float32)
            acc = acc_h[:br] + acc_l[:br] * (1.0 / 16.0)
            mean = acc * jnp.transpose(s_scr[j])
            xr = x1_scr[pl.ds(j * br, br), :]
            out, miss = _relation(xr, mean, wx2_ref, wm2_ref, m2_ref,
                                  w2_ref, fac_ref[0])
            x2_ref[...] = out
            miss2_ref[...] = miss
            mx = jnp.max(out, axis=1, keepdims=True)
            sh = out - mx
            lse = jnp.log(jnp.sum(jnp.exp(sh), axis=1, keepdims=True))
            lsm_ref[...] = sh - lse

        @pl.when(j % 2 == 0)
        def _use_even():
            pltpu.make_async_copy(q_stripe(j), qb0, rd_sem0).wait()
            compute(qb0)

        @pl.when(j % 2 == 1)
        def _use_odd():
            pltpu.make_async_copy(q_stripe(j), qb1, rd_sem1).wait()
            compute(qb1)


def kernel(x, adj, head, r1_g1, r1_g2, r1_b1, r1_b2, r2_g1, r2_g2, r2_b1,
           r2_b2, r1_m, r2_m, r1_w, r2_w):
    n, f = x.shape
    fo = r2_w.shape[1]
    br = next(b for b in (400, 200, 80, 16, 8, 1) if n % b == 0)
    brp = ((br + 31) // 32) * 32
    g = n // br
    fac = jnp.where(head != 0, 0.0, G_SIGMA).astype(jnp.float32).reshape(1)
    wx1 = jnp.concatenate([r1_g1, r1_b1], axis=1)
    wm1 = jnp.concatenate([r1_g2, r1_b2], axis=1)
    wx2 = jnp.concatenate([r2_g1, r2_b1], axis=1)
    wm2 = jnp.concatenate([r2_g2, r2_b2], axis=1)

    res = pl.pallas_call(
        functools.partial(_fused_body, br, brp, g),
        grid=(2 * g,),
        in_specs=[
            pl.BlockSpec((br, n), lambda i: (jnp.minimum(i, g - 1), 0)),
            pl.BlockSpec((n, f), lambda i: (0, 0)),
            pl.BlockSpec((f, 2 * f), lambda i: (0, 0)),
            pl.BlockSpec((f, 2 * f), lambda i: (0, 0)),
            pl.BlockSpec((1, f), lambda i: (0, 0)),
            pl.BlockSpec((f, f), lambda i: (0, 0)),
            pl.BlockSpec((f, 2 * f), lambda i: (0, 0)),
            pl.BlockSpec((f, 2 * f), lambda i: (0, 0)),
            pl.BlockSpec((1, f), lambda i: (0, 0)),
            pl.BlockSpec((f, fo), lambda i: (0, 0)),
            pl.BlockSpec(memory_space=pltpu.SMEM),
        ],
        out_specs=[
            pl.BlockSpec((br, fo), lambda i: (jnp.maximum(i - g, 0), 0)),
            pl.BlockSpec((br, fo), lambda i: (jnp.maximum(i - g, 0), 0)),
            pl.BlockSpec((br, f), lambda i: (jnp.minimum(i, g - 1), 0)),
            pl.BlockSpec((br, f), lambda i: (jnp.maximum(i - g, 0), 0)),
            pl.BlockSpec(memory_space=pl.ANY),
        ],
        out_shape=[
            jax.ShapeDtypeStruct((n, fo), jnp.float32),
            jax.ShapeDtypeStruct((n, fo), jnp.float32),
            jax.ShapeDtypeStruct((n, f), jnp.float32),
            jax.ShapeDtypeStruct((n, f), jnp.float32),
            jax.ShapeDtypeStruct((g * brp, n), jnp.float8_e4m3fn),
        ],
        scratch_shapes=[
            pltpu.VMEM((brp, n), jnp.float8_e4m3fn),
            pltpu.VMEM((brp, n), jnp.float8_e4m3fn),
            pltpu.VMEM((n, f), jnp.float32),
            pltpu.VMEM((g, 1, br), jnp.float32),
            pltpu.VMEM((n, f), jnp.float8_e4m3fn),
            pltpu.VMEM((n, f), jnp.float8_e4m3fn),
            pltpu.SemaphoreType.DMA,
            pltpu.SemaphoreType.DMA,
            pltpu.SemaphoreType.DMA,
            pltpu.SemaphoreType.DMA,
        ],
        compiler_params=pltpu.CompilerParams(
            dimension_semantics=("arbitrary",),
            vmem_limit_bytes=64 * 1024 * 1024,
        ),
    )(adj, x, wx1, wm1, r1_m, r1_w, wx2, wm2, r2_m, r2_w, fac)
    x2, lsm, out1, out2, _ = res
    return x2, lsm, out1, out2


# speedup vs baseline: 1.1225x; 1.0984x over previous
import functools

import jax
import jax.numpy as jnp
from jax.experimental import pallas as pl
from jax.experimental.pallas import tpu as pltpu

G_SIGMA = 1.0
_C = 256.0


def _lrelu(v):
    return jnp.where(v >= 0, v, 0.2 * v)


def _elu(v):
    return jnp.where(v > 0, v, jnp.exp(v) - 1.0)


def _relation(xr, mean, wx_ref, wm_ref, m_ref, w_ref, fac):
    f = xr.shape[1]
    gb = (jnp.dot(xr, wx_ref[...], preferred_element_type=jnp.float32)
          + jnp.dot(mean, wm_ref[...], preferred_element_type=jnp.float32))
    gamma = _lrelu(gb[:, :f]) + 1.0
    beta = _lrelu(gb[:, f:])
    miss = xr + gamma * m_ref[...] + beta - mean
    h = mean + fac * miss
    out = jnp.dot(h, w_ref[...], preferred_element_type=jnp.float32)
    return out, miss


def _fused_body(br, brp, g,
                adj_ref, x_ref, wx1_ref, wm1_ref, m1_ref, w1_ref,
                wx2_ref, wm2_ref, m2_ref, w2_ref, fac_ref,
                x2_ref, lsm_ref, miss1_ref, miss2_ref, q_ref,
                qb0, qb1, x1_scr, s_scr, hl_ref,
                st_sem0, st_sem1, rd_sem0, rd_sem1):
    i = pl.program_id(0)
    n = adj_ref.shape[1]
    f = x_ref.shape[1]

    def q_stripe(k):
        return q_ref.at[pl.ds(k * brp, brp), :]

    @pl.when(i < g)
    def _phase_a():
        adjb = adj_ref[...]
        mean = jnp.dot(adjb, x_ref[...], preferred_element_type=jnp.float32)
        rmax = jnp.maximum(jnp.max(jnp.abs(adjb), axis=1, keepdims=True),
                           1e-30)
        scaled = jnp.concatenate(
            [adjb * (_C / rmax), jnp.zeros((brp - br, n), jnp.float32)],
            axis=0)
        qv = scaled.astype(jnp.float8_e4m3fn)
        s_scr[i] = jnp.transpose(rmax * (1.0 / _C))

        @pl.when(i % 2 == 0)
        def _even():
            @pl.when(i >= 2)
            def _():
                pltpu.make_async_copy(qb0, q_stripe(i - 2), st_sem0).wait()
            qb0[...] = qv
            pltpu.make_async_copy(qb0, q_stripe(i), st_sem0).start()

        @pl.when(i % 2 == 1)
        def _odd():
            @pl.when(i >= 2)
            def _():
                pltpu.make_async_copy(qb1, q_stripe(i - 2), st_sem1).wait()
            qb1[...] = qv
            pltpu.make_async_copy(qb1, q_stripe(i), st_sem1).start()

        xr = x_ref[pl.ds(i * br, br), :]
        out, miss = _relation(xr, mean, wx1_ref, wm1_ref, m1_ref, w1_ref,
                              fac_ref[0])
        miss1_ref[...] = miss
        x1_scr[pl.ds(i * br, br), :] = _elu(out)

    @pl.when(i >= g)
    def _phase_b():
        j = i - g

        @pl.when(j == 0)
        def _start_b():
            pltpu.make_async_copy(qb0, q_stripe(0), st_sem0).wait()
            if g >= 2:
                pltpu.make_async_copy(qb1, q_stripe(0), st_sem1).wait()
            v = x1_scr[...]
            sx = jnp.maximum(jnp.max(jnp.abs(v)), 1e-30) * (1.0 / _C)
            vi = v * (1.0 / sx)
            hq = vi.astype(jnp.float8_e4m3fn)
            lq = ((vi - hq.astype(jnp.float32)) * 16.0).astype(
                jnp.float8_e4m3fn)
            hl_ref[...] = jnp.concatenate([hq, lq], axis=1)
            s_scr[...] = s_scr[...] * sx
            pltpu.make_async_copy(q_stripe(0), qb0, rd_sem0).start()
            if g >= 2:
                pltpu.make_async_copy(q_stripe(1), qb1, rd_sem1).start()

        @pl.when((j > 0) & (j + 1 < g) & ((j + 1) % 2 == 0))
        def _pf_even():
            pltpu.make_async_copy(q_stripe(j + 1), qb0, rd_sem0).start()

        @pl.when((j > 0) & (j + 1 < g) & ((j + 1) % 2 == 1))
        def _pf_odd():
            pltpu.make_async_copy(q_stripe(j + 1), qb1, rd_sem1).start()

        def compute(qb_ref):
            qa = qb_ref[...]
            f = x1_scr.shape[1]
            acc2 = jnp.dot(qa, hl_ref[...],
                           preferred_element_type=jnp.float32)
            acc = acc2[:br, :f] + acc2[:br, f:] * (1.0 / 16.0)
            mean = acc * jnp.transpose(s_scr[j])
            xr = x1_scr[pl.ds(j * br, br), :]
            out, miss = _relation(xr, mean, wx2_ref, wm2_ref, m2_ref,
                                  w2_ref, fac_ref[0])
            x2_ref[...] = out
            miss2_ref[...] = miss
            mx = jnp.max(out, axis=1, keepdims=True)
            sh = out - mx
            lse = jnp.log(jnp.sum(jnp.exp(sh), axis=1, keepdims=True))
            lsm_ref[...] = sh - lse

        @pl.when(j % 2 == 0)
        def _use_even():
            pltpu.make_async_copy(q_stripe(j), qb0, rd_sem0).wait()
            compute(qb0)

        @pl.when(j % 2 == 1)
        def _use_odd():
            pltpu.make_async_copy(q_stripe(j), qb1, rd_sem1).wait()
            compute(qb1)


def kernel(x, adj, head, r1_g1, r1_g2, r1_b1, r1_b2, r2_g1, r2_g2, r2_b1,
           r2_b2, r1_m, r2_m, r1_w, r2_w):
    n, f = x.shape
    fo = r2_w.shape[1]
    br = next(b for b in (400, 200, 80, 16, 8, 1) if n % b == 0)
    brp = ((br + 31) // 32) * 32
    g = n // br
    fac = jnp.where(head != 0, 0.0, G_SIGMA).astype(jnp.float32).reshape(1)
    wx1 = jnp.concatenate([r1_g1, r1_b1], axis=1)
    wm1 = jnp.concatenate([r1_g2, r1_b2], axis=1)
    wx2 = jnp.concatenate([r2_g1, r2_b1], axis=1)
    wm2 = jnp.concatenate([r2_g2, r2_b2], axis=1)

    res = pl.pallas_call(
        functools.partial(_fused_body, br, brp, g),
        grid=(2 * g,),
        in_specs=[
            pl.BlockSpec((br, n), lambda i: (jnp.minimum(i, g - 1), 0)),
            pl.BlockSpec((n, f), lambda i: (0, 0)),
            pl.BlockSpec((f, 2 * f), lambda i: (0, 0)),
            pl.BlockSpec((f, 2 * f), lambda i: (0, 0)),
            pl.BlockSpec((1, f), lambda i: (0, 0)),
            pl.BlockSpec((f, f), lambda i: (0, 0)),
            pl.BlockSpec((f, 2 * f), lambda i: (0, 0)),
            pl.BlockSpec((f, 2 * f), lambda i: (0, 0)),
            pl.BlockSpec((1, f), lambda i: (0, 0)),
            pl.BlockSpec((f, fo), lambda i: (0, 0)),
            pl.BlockSpec(memory_space=pltpu.SMEM),
        ],
        out_specs=[
            pl.BlockSpec((br, fo), lambda i: (jnp.maximum(i - g, 0), 0)),
            pl.BlockSpec((br, fo), lambda i: (jnp.maximum(i - g, 0), 0)),
            pl.BlockSpec((br, f), lambda i: (jnp.minimum(i, g - 1), 0)),
            pl.BlockSpec((br, f), lambda i: (jnp.maximum(i - g, 0), 0)),
            pl.BlockSpec(memory_space=pl.ANY),
        ],
        out_shape=[
            jax.ShapeDtypeStruct((n, fo), jnp.float32),
            jax.ShapeDtypeStruct((n, fo), jnp.float32),
            jax.ShapeDtypeStruct((n, f), jnp.float32),
            jax.ShapeDtypeStruct((n, f), jnp.float32),
            jax.ShapeDtypeStruct((g * brp, n), jnp.float8_e4m3fn),
        ],
        scratch_shapes=[
            pltpu.VMEM((brp, n), jnp.float8_e4m3fn),
            pltpu.VMEM((brp, n), jnp.float8_e4m3fn),
            pltpu.VMEM((n, f), jnp.float32),
            pltpu.VMEM((g, 1, br), jnp.float32),
            pltpu.VMEM((n, 2 * f), jnp.float8_e4m3fn),
            pltpu.SemaphoreType.DMA,
            pltpu.SemaphoreType.DMA,
            pltpu.SemaphoreType.DMA,
            pltpu.SemaphoreType.DMA,
        ],
        compiler_params=pltpu.CompilerParams(
            dimension_semantics=("arbitrary",),
            vmem_limit_bytes=64 * 1024 * 1024,
        ),
    )(adj, x, wx1, wm1, r1_m, r1_w, wx2, wm2, r2_m, r2_w, fac)
    x2, lsm, out1, out2, _ = res
    return x2, lsm, out1, out2
